# baseline (speedup 1.0000x reference)
import functools

import jax
import jax.numpy as jnp
from jax import lax
from jax.experimental import pallas as pl
from jax.experimental.pallas import tpu as pltpu
from jax.experimental.pallas import tpu_sc as plsc

N = 10000
H = 256
HALF = 128
E = 160000
NS = 16
NC = 2
BATCH = 128
BPT = 80
EB = NS * BPT
EPAD = EB * BATCH - E
CHB = 16
NCH = BPT // CHB
STRIPE = 624
NREM = N - NS * STRIPE
ZROWS = 16
RBLK = 2000
GRID = N // RBLK

_SC_MESH = dict(core_axis_name="c", subcore_axis_name="s")


def _zero_fill(buf, nrows, ncols):
    zeros16 = jnp.zeros((16,), jnp.float32)

    def body(i, _):
        for j in range(ncols // 16):
            buf[i, pl.ds(j * 16, 16)] = zeros16
        return 0

    lax.fori_loop(0, nrows, body, 0)


def _zero_acc(acc, s, zbuf):
    base = s * STRIPE

    def body(k, _):
        pltpu.sync_copy(zbuf, acc.at[pl.ds(base + k * 16, 16)])
        return 0

    lax.fori_loop(0, STRIPE // 16, body, 0)

    @pl.when(s == NS - 1)
    def _():
        pltpu.sync_copy(zbuf, acc.at[pl.ds(NS * STRIPE, NREM)])


def _writeout(acc, out_ref, s):
    base = s * STRIPE
    pltpu.sync_copy(acc.at[pl.ds(base, STRIPE)],
                    out_ref.at[pl.ds(base, STRIPE)])

    @pl.when(s == NS - 1)
    def _():
        pltpu.sync_copy(acc.at[pl.ds(NS * STRIPE, NREM)],
                        out_ref.at[pl.ds(NS * STRIPE, NREM)])


def _graph_pass(gather_tab, row2d, col2d, w1d, out_ref, ncols,
                rowv2, colv2, wv, rbufA, rbufB, zbuf, acc, semA, semB,
                semSA, semSB, s):
    base_b = s * BPT
    plsc.subcore_barrier()

    def scale_rows(rbuf, b):
        boff = b * BATCH

        def scale(g, _):
            wvec = wv[pl.ds(boff + g * 16, 16)]
            for t in range(16):
                e = g * 16 + t
                for j in range(ncols // 16):
                    sl = pl.ds(j * 16, 16)
                    rbuf[e, sl] = rbuf[e, sl] * wvec[t]
            return 0

        lax.fori_loop(0, BATCH // 16, scale, 0)

    def chunk(ch, _):
        cb = base_b + ch * CHB
        pltpu.sync_copy(row2d.at[pl.ds(cb, CHB)], rowv2)
        pltpu.sync_copy(col2d.at[pl.ds(cb, CHB)], colv2)
        pltpu.sync_copy(w1d.at[pl.ds(cb * BATCH, CHB * BATCH)], wv)

        pltpu.async_copy(gather_tab.at[rowv2.at[0]], rbufA, semA)

        def pair(i, _):
            b0 = 2 * i

            @pl.when(i > 0)
            def _():
                bprev = jnp.maximum(b0 - 1, 0)
                pltpu.make_async_copy(rbufB, acc.at[colv2.at[bprev]],
                                      semSB).wait()

            pltpu.async_copy(gather_tab.at[rowv2.at[b0 + 1]], rbufB, semB)
            pltpu.make_async_copy(gather_tab.at[rowv2.at[b0]],
                                  rbufA, semA).wait()
            scale_rows(rbufA, b0)
            pltpu.async_copy(rbufA, acc.at[colv2.at[b0]], semSA, add=True)
            pltpu.make_async_copy(gather_tab.at[rowv2.at[b0 + 1]],
                                  rbufB, semB).wait()
            scale_rows(rbufB, b0 + 1)
            pltpu.make_async_copy(rbufA, acc.at[colv2.at[b0]], semSA).wait()
            nxt = jnp.minimum(b0 + 2, CHB - 1)
            pltpu.async_copy(gather_tab.at[rowv2.at[nxt]], rbufA, semA)
            pltpu.async_copy(rbufB, acc.at[colv2.at[b0 + 1]], semSB,
                             add=True)
            return 0

        lax.fori_loop(0, CHB // 2, pair, 0)
        pltpu.make_async_copy(rbufB, acc.at[colv2.at[CHB - 1]],
                              semSB).wait()
        pltpu.make_async_copy(gather_tab.at[rowv2.at[0]], rbufA, semA).wait()
        return 0

    lax.fori_loop(0, NCH, chunk, 0)

    plsc.subcore_barrier()
    _writeout(acc, out_ref, s)
    plsc.subcore_barrier()


def _deg_body(colu2d, wu1d, colg2d, wg1d, ones_tab, out,
              rowv2, colv2, wv, rbufA, rbufB, zbuf, acc, semA, semB,
              semSA, semSB):
    c = lax.axis_index("c")
    s = lax.axis_index("s")
    _zero_fill(zbuf, ZROWS, HALF)
    _zero_acc(acc, s, zbuf)

    @pl.when(c == 0)
    def _():
        _graph_pass(ones_tab, colu2d, colu2d, wu1d, out.at[0], HALF,
                    rowv2, colv2, wv, rbufA, rbufB, zbuf, acc, semA, semB,
                    semSA, semSB, s)

    @pl.when(c == 1)
    def _():
        _graph_pass(ones_tab, colg2d, colg2d, wg1d, out.at[1], HALF,
                    rowv2, colv2, wv, rbufA, rbufB, zbuf, acc, semA, semB,
                    semSA, semSB, s)


def _deg_sc(col_u2d, w_u1d, col_g2d, w_g1d, ones_tab):
    return pl.kernel(
        _deg_body,
        out_type=jax.ShapeDtypeStruct((2, N, HALF), jnp.float32),
        mesh=plsc.VectorSubcoreMesh(**_SC_MESH),
        scratch_types=_mp_scratch(),
    )(col_u2d, w_u1d, col_g2d, w_g1d, ones_tab)


def _mp_core(y0, y1, row2d, col2d, w1d, out0, out1,
             rowv2, colv2, wv, rbufA, rbufB, zbuf, acc, semA, semB,
             semSA, semSB, c, s):
    _zero_acc(acc, s, zbuf)

    @pl.when(c == 0)
    def _():
        _graph_pass(y0, row2d, col2d, w1d, out0, HALF,
                    rowv2, colv2, wv, rbufA, rbufB, zbuf, acc, semA, semB,
                    semSA, semSB, s)

    @pl.when(c == 1)
    def _():
        _graph_pass(y1, row2d, col2d, w1d, out1, HALF,
                    rowv2, colv2, wv, rbufA, rbufB, zbuf, acc, semA, semB,
                    semSA, semSB, s)


def _mp_body_2(yu0, yu1, rowu, colu, wu, yg0, yg1, rowg, colg, wg,
               ou0, ou1, og0, og1, rowv2, colv2, wv, rbufA, rbufB, zbuf,
               acc, semA, semB, semSA, semSB):
    c = lax.axis_index("c")
    s = lax.axis_index("s")
    _zero_fill(zbuf, ZROWS, HALF)
    _mp_core(yu0, yu1, rowu, colu, wu, ou0, ou1,
             rowv2, colv2, wv, rbufA, rbufB, zbuf, acc, semA, semB,
             semSA, semSB, c, s)
    _mp_core(yg0, yg1, rowg, colg, wg, og0, og1,
             rowv2, colv2, wv, rbufA, rbufB, zbuf, acc, semA, semB,
             semSA, semSB, c, s)


def _mp_body_1(yu0, yu1, rowu, colu, wu, ou0, ou1,
               rowv2, colv2, wv, rbufA, rbufB, zbuf, acc, semA, semB,
               semSA, semSB):
    c = lax.axis_index("c")
    s = lax.axis_index("s")
    _zero_fill(zbuf, ZROWS, HALF)
    _mp_core(yu0, yu1, rowu, colu, wu, ou0, ou1,
             rowv2, colv2, wv, rbufA, rbufB, zbuf, acc, semA, semB,
             semSA, semSB, c, s)


def _mp_scratch():
    return [
        pltpu.VMEM((CHB, BATCH), jnp.int32),
        pltpu.VMEM((CHB, BATCH), jnp.int32),
        pltpu.VMEM((CHB * BATCH,), jnp.float32),
        pltpu.VMEM((BATCH, HALF), jnp.float32),
        pltpu.VMEM((BATCH, HALF), jnp.float32),
        pltpu.VMEM((ZROWS, HALF), jnp.float32),
        pltpu.VMEM_SHARED((N, HALF), jnp.float32),
        pltpu.SemaphoreType.DMA,
        pltpu.SemaphoreType.DMA,
        pltpu.SemaphoreType.DMA,
        pltpu.SemaphoreType.DMA,
    ]


def _mp_sc2(yu0, yu1, rowu, colu, wu, yg0, yg1, rowg, colg, wg):
    half = jax.ShapeDtypeStruct((N, HALF), jnp.float32)
    return pl.kernel(
        _mp_body_2,
        out_type=(half, half, half, half),
        mesh=plsc.VectorSubcoreMesh(**_SC_MESH),
        scratch_types=_mp_scratch(),
    )(yu0, yu1, rowu, colu, wu, yg0, yg1, rowg, colg, wg)


def _mp_sc1(y0, y1, row, col, w):
    half = jax.ShapeDtypeStruct((N, HALF), jnp.float32)
    return pl.kernel(
        _mp_body_1,
        out_type=(half, half),
        mesh=plsc.VectorSubcoreMesh(**_SC_MESH),
        scratch_types=_mp_scratch(),
    )(y0, y1, row, col, w)


def _dis_of(degblk):
    d = degblk + 1.0
    return jnp.where(d > 0, lax.rsqrt(jnp.maximum(d, 1e-12)), 0.0)


def _tc_b_body(deg_ref, embu_ref, wu_ref, embg_ref, wg_ref,
               yu0_ref, yu1_ref, disu_ref, yg0_ref, yg1_ref, disg_ref):
    disu = _dis_of(deg_ref[0][:, 0:1])
    disg = _dis_of(deg_ref[1][:, 0:1])
    yu = disu * jnp.dot(embu_ref[...], wu_ref[...],
                        preferred_element_type=jnp.float32)
    yg = disg * jnp.dot(embg_ref[...], wg_ref[...],
                        preferred_element_type=jnp.float32)
    yu0_ref[...] = yu[:, :HALF]
    yu1_ref[...] = yu[:, HALF:]
    disu_ref[...] = disu
    yg0_ref[...] = yg[:, :HALF]
    yg1_ref[...] = yg[:, HALF:]
    disg_ref[...] = disg


def _tc_b(deg2, emb_u, W_u1, emb_g, W_g1):
    half = jax.ShapeDtypeStruct((N, HALF), jnp.float32)
    dis = jax.ShapeDtypeStruct((N, 1), jnp.float32)
    return pl.pallas_call(
        _tc_b_body,
        grid=(GRID,),
        in_specs=[
            pl.BlockSpec((2, RBLK, HALF), lambda i: (0, i, 0)),
            pl.BlockSpec((RBLK, H), lambda i: (i, 0)),
            pl.BlockSpec((H, H), lambda i: (0, 0)),
            pl.BlockSpec((RBLK, H), lambda i: (i, 0)),
            pl.BlockSpec((H, H), lambda i: (0, 0)),
        ],
        out_specs=[
            pl.BlockSpec((RBLK, HALF), lambda i: (i, 0)),
            pl.BlockSpec((RBLK, HALF), lambda i: (i, 0)),
            pl.BlockSpec((RBLK, 1), lambda i: (i, 0)),
            pl.BlockSpec((RBLK, HALF), lambda i: (i, 0)),
            pl.BlockSpec((RBLK, HALF), lambda i: (i, 0)),
            pl.BlockSpec((RBLK, 1), lambda i: (i, 0)),
        ],
        out_shape=(half, half, dis, half, half, dis),
    )(deg2, emb_u, W_u1, emb_g, W_g1)


def _tc_d_body(au0_ref, au1_ref, yu0_ref, yu1_ref, disu_ref, bu_ref, wu2_ref,
               ag0_ref, ag1_ref, yg0_ref, yg1_ref, disg_ref, bg_ref,
               g_ref, y20_ref, y21_ref):
    disu = disu_ref[...]
    u1 = jnp.concatenate(
        [au0_ref[...] + yu0_ref[...], au1_ref[...] + yu1_ref[...]], axis=1)
    u1 = jax.nn.relu(disu * u1 + bu_ref[...])
    y2 = disu * jnp.dot(u1, wu2_ref[...], preferred_element_type=jnp.float32)
    y20_ref[...] = y2[:, :HALF]
    y21_ref[...] = y2[:, HALF:]
    disg = disg_ref[...]
    g = jnp.concatenate(
        [ag0_ref[...] + yg0_ref[...], ag1_ref[...] + yg1_ref[...]], axis=1)
    g_ref[...] = jax.nn.relu(disg * g + bg_ref[...])


def _tc_d(au0, au1, yu0, yu1, dis_u, b_u1, W_u2, ag0, ag1, yg0, yg1, dis_g,
          b_g1):
    half = jax.ShapeDtypeStruct((N, HALF), jnp.float32)
    full = jax.ShapeDtypeStruct((N, H), jnp.float32)
    rb = lambda i: (i, 0)
    return pl.pallas_call(
        _tc_d_body,
        grid=(GRID,),
        in_specs=[
            pl.BlockSpec((RBLK, HALF), rb), pl.BlockSpec((RBLK, HALF), rb),
            pl.BlockSpec((RBLK, HALF), rb), pl.BlockSpec((RBLK, HALF), rb),
            pl.BlockSpec((RBLK, 1), rb),
            pl.BlockSpec((1, H), lambda i: (0, 0)),
            pl.BlockSpec((H, H), lambda i: (0, 0)),
            pl.BlockSpec((RBLK, HALF), rb), pl.BlockSpec((RBLK, HALF), rb),
            pl.BlockSpec((RBLK, HALF), rb), pl.BlockSpec((RBLK, HALF), rb),
            pl.BlockSpec((RBLK, 1), rb),
            pl.BlockSpec((1, H), lambda i: (0, 0)),
        ],
        out_specs=[
            pl.BlockSpec((RBLK, H), rb),
            pl.BlockSpec((RBLK, HALF), rb), pl.BlockSpec((RBLK, HALF), rb),
        ],
        out_shape=(full, half, half),
    )(au0, au1, yu0, yu1, dis_u, b_u1, W_u2, ag0, ag1, yg0, yg1, dis_g, b_g1)


def _tc_f_body(a0_ref, a1_ref, y0_ref, y1_ref, dis_ref, b_ref, u_ref):
    u = jnp.concatenate(
        [a0_ref[...] + y0_ref[...], a1_ref[...] + y1_ref[...]], axis=1)
    u_ref[...] = dis_ref[...] * u + b_ref[...]


def _tc_f(a0, a1, y0, y1, dis, b):
    rb = lambda i: (i, 0)
    return pl.pallas_call(
        _tc_f_body,
        grid=(GRID,),
        in_specs=[
            pl.BlockSpec((RBLK, HALF), rb), pl.BlockSpec((RBLK, HALF), rb),
            pl.BlockSpec((RBLK, HALF), rb), pl.BlockSpec((RBLK, HALF), rb),
            pl.BlockSpec((RBLK, 1), rb),
            pl.BlockSpec((1, H), lambda i: (0, 0)),
        ],
        out_specs=pl.BlockSpec((RBLK, H), rb),
        out_shape=jax.ShapeDtypeStruct((N, H), jnp.float32),
    )(a0, a1, y0, y1, dis, b)


def kernel(x, user_edge_index, user_edge_weight, group_edge_index,
           group_edge_weight, user_emb, group_emb, W_u1, b_u1, W_u2, b_u2,
           W_g1, b_g1):
    ipad = jnp.zeros((EPAD,), user_edge_index.dtype)
    fpad = jnp.zeros((EPAD,), jnp.float32)
    row_u = jnp.concatenate([user_edge_index[0], ipad]).reshape(EB, BATCH)
    col_u = jnp.concatenate([user_edge_index[1], ipad]).reshape(EB, BATCH)
    row_g = jnp.concatenate([group_edge_index[0], ipad]).reshape(EB, BATCH)
    col_g = jnp.concatenate([group_edge_index[1], ipad]).reshape(EB, BATCH)
    w_u = jnp.concatenate([user_edge_weight, fpad])
    w_g = jnp.concatenate([group_edge_weight, fpad])

    ones_tab = jnp.ones((N, HALF), jnp.float32)
    deg2 = _deg_sc(col_u, w_u, col_g, w_g, ones_tab)
    yu0, yu1, dis_u, yg0, yg1, dis_g = _tc_b(deg2, user_emb, W_u1,
                                             group_emb, W_g1)
    au0, au1, ag0, ag1 = _mp_sc2(yu0, yu1, row_u, col_u, w_u,
                                 yg0, yg1, row_g, col_g, w_g)
    g, y20, y21 = _tc_d(au0, au1, yu0, yu1, dis_u, b_u1.reshape(1, H), W_u2,
                        ag0, ag1, yg0, yg1, dis_g, b_g1.reshape(1, H))
    a20, a21 = _mp_sc1(y20, y21, row_u, col_u, w_u)
    u = _tc_f(a20, a21, y20, y21, dis_u, b_u2.reshape(1, H))
    return (u, g)


# speedup vs baseline: 1.0444x; 1.0444x over previous
import functools

import jax
import jax.numpy as jnp
from jax import lax
from jax.experimental import pallas as pl
from jax.experimental.pallas import tpu as pltpu
from jax.experimental.pallas import tpu_sc as plsc

N = 10000
H = 256
HALF = 128
E = 160000
NS = 16
NC = 2
BATCH = 128
BPT = 80
EB = NS * BPT
EPAD = EB * BATCH - E
CHB = 16
NCH = BPT // CHB
STRIPE = 624
NREM = N - NS * STRIPE
ZROWS = 16
RBLK = 2000
GRID = N // RBLK

_SC_MESH = dict(core_axis_name="c", subcore_axis_name="s")


def _zero_fill(buf, nrows, ncols):
    zeros16 = jnp.zeros((16,), jnp.float32)

    def body(i, _):
        for j in range(ncols // 16):
            buf[i, pl.ds(j * 16, 16)] = zeros16
        return 0

    lax.fori_loop(0, nrows, body, 0)


def _zero_acc(acc, s, zbuf):
    base = s * STRIPE

    def body(k, _):
        pltpu.sync_copy(zbuf, acc.at[pl.ds(base + k * 16, 16)])
        return 0

    lax.fori_loop(0, STRIPE // 16, body, 0)

    @pl.when(s == NS - 1)
    def _():
        pltpu.sync_copy(zbuf, acc.at[pl.ds(NS * STRIPE, NREM)])


def _writeout(acc, out_ref, s):
    base = s * STRIPE
    pltpu.sync_copy(acc.at[pl.ds(base, STRIPE)],
                    out_ref.at[pl.ds(base, STRIPE)])

    @pl.when(s == NS - 1)
    def _():
        pltpu.sync_copy(acc.at[pl.ds(NS * STRIPE, NREM)],
                        out_ref.at[pl.ds(NS * STRIPE, NREM)])


def _graph_pass(gather_tab, row2d, col2d, w1d, out_ref, ncols,
                rowv2, colv2, wv, rbufA, rbufB, zbuf, acc, semA, semB,
                semSA, semSB, s):
    base_b = s * BPT
    plsc.subcore_barrier()

    def scale_rows(rbuf, b):
        boff = b * BATCH

        zeros16 = jnp.zeros((16,), jnp.float32)

        def scale(g, _):
            wvec = wv[pl.ds(boff + g * 16, 16)]
            for t in range(16):
                e = g * 16 + t
                wrow = zeros16 + wvec[t]
                for j in range(ncols // 16):
                    sl = pl.ds(j * 16, 16)
                    rbuf[e, sl] = rbuf[e, sl] * wrow
            return 0

        lax.fori_loop(0, BATCH // 16, scale, 0)

    def chunk(ch, _):
        cb = base_b + ch * CHB
        pltpu.sync_copy(row2d.at[pl.ds(cb, CHB)], rowv2)
        pltpu.sync_copy(col2d.at[pl.ds(cb, CHB)], colv2)
        pltpu.sync_copy(w1d.at[pl.ds(cb * BATCH, CHB * BATCH)], wv)

        pltpu.async_copy(gather_tab.at[rowv2.at[0]], rbufA, semA)

        def pair(i, _):
            b0 = 2 * i
            pltpu.async_copy(gather_tab.at[rowv2.at[b0 + 1]], rbufB, semB)
            pltpu.make_async_copy(gather_tab.at[rowv2.at[b0]],
                                  rbufA, semA).wait()
            scale_rows(rbufA, b0)
            pltpu.sync_copy(rbufA, acc.at[colv2.at[b0]], add=True)
            nxt = jnp.minimum(b0 + 2, CHB - 1)
            pltpu.async_copy(gather_tab.at[rowv2.at[nxt]], rbufA, semA)
            pltpu.make_async_copy(gather_tab.at[rowv2.at[b0 + 1]],
                                  rbufB, semB).wait()
            scale_rows(rbufB, b0 + 1)
            pltpu.sync_copy(rbufB, acc.at[colv2.at[b0 + 1]], add=True)
            return 0

        lax.fori_loop(0, CHB // 2, pair, 0)
        pltpu.make_async_copy(gather_tab.at[rowv2.at[0]], rbufA, semA).wait()
        return 0

    lax.fori_loop(0, NCH, chunk, 0)

    plsc.subcore_barrier()
    _writeout(acc, out_ref, s)
    plsc.subcore_barrier()


def _deg_body(colu2d, wu1d, colg2d, wg1d, ones_tab, out,
              rowv2, colv2, wv, rbufA, rbufB, zbuf, acc, semA, semB,
              semSA, semSB):
    c = lax.axis_index("c")
    s = lax.axis_index("s")
    _zero_fill(zbuf, ZROWS, HALF)
    _zero_acc(acc, s, zbuf)

    @pl.when(c == 0)
    def _():
        _graph_pass(ones_tab, colu2d, colu2d, wu1d, out.at[0], HALF,
                    rowv2, colv2, wv, rbufA, rbufB, zbuf, acc, semA, semB,
                    semSA, semSB, s)

    @pl.when(c == 1)
    def _():
        _graph_pass(ones_tab, colg2d, colg2d, wg1d, out.at[1], HALF,
                    rowv2, colv2, wv, rbufA, rbufB, zbuf, acc, semA, semB,
                    semSA, semSB, s)


def _deg_sc(col_u2d, w_u1d, col_g2d, w_g1d, ones_tab):
    return pl.kernel(
        _deg_body,
        out_type=jax.ShapeDtypeStruct((2, N, HALF), jnp.float32),
        mesh=plsc.VectorSubcoreMesh(**_SC_MESH),
        scratch_types=_mp_scratch(),
    )(col_u2d, w_u1d, col_g2d, w_g1d, ones_tab)


def _mp_core(y0, y1, row2d, col2d, w1d, out0, out1,
             rowv2, colv2, wv, rbufA, rbufB, zbuf, acc, semA, semB,
             semSA, semSB, c, s):
    _zero_acc(acc, s, zbuf)

    @pl.when(c == 0)
    def _():
        _graph_pass(y0, row2d, col2d, w1d, out0, HALF,
                    rowv2, colv2, wv, rbufA, rbufB, zbuf, acc, semA, semB,
                    semSA, semSB, s)

    @pl.when(c == 1)
    def _():
        _graph_pass(y1, row2d, col2d, w1d, out1, HALF,
                    rowv2, colv2, wv, rbufA, rbufB, zbuf, acc, semA, semB,
                    semSA, semSB, s)


def _mp_body_2(yu0, yu1, rowu, colu, wu, yg0, yg1, rowg, colg, wg,
               ou0, ou1, og0, og1, rowv2, colv2, wv, rbufA, rbufB, zbuf,
               acc, semA, semB, semSA, semSB):
    c = lax.axis_index("c")
    s = lax.axis_index("s")
    _zero_fill(zbuf, ZROWS, HALF)
    _mp_core(yu0, yu1, rowu, colu, wu, ou0, ou1,
             rowv2, colv2, wv, rbufA, rbufB, zbuf, acc, semA, semB,
             semSA, semSB, c, s)
    _mp_core(yg0, yg1, rowg, colg, wg, og0, og1,
             rowv2, colv2, wv, rbufA, rbufB, zbuf, acc, semA, semB,
             semSA, semSB, c, s)


def _mp_body_1(yu0, yu1, rowu, colu, wu, ou0, ou1,
               rowv2, colv2, wv, rbufA, rbufB, zbuf, acc, semA, semB,
               semSA, semSB):
    c = lax.axis_index("c")
    s = lax.axis_index("s")
    _zero_fill(zbuf, ZROWS, HALF)
    _mp_core(yu0, yu1, rowu, colu, wu, ou0, ou1,
             rowv2, colv2, wv, rbufA, rbufB, zbuf, acc, semA, semB,
             semSA, semSB, c, s)


def _mp_scratch():
    return [
        pltpu.VMEM((CHB, BATCH), jnp.int32),
        pltpu.VMEM((CHB, BATCH), jnp.int32),
        pltpu.VMEM((CHB * BATCH,), jnp.float32),
        pltpu.VMEM((BATCH, HALF), jnp.float32),
        pltpu.VMEM((BATCH, HALF), jnp.float32),
        pltpu.VMEM((ZROWS, HALF), jnp.float32),
        pltpu.VMEM_SHARED((N, HALF), jnp.float32),
        pltpu.SemaphoreType.DMA,
        pltpu.SemaphoreType.DMA,
        pltpu.SemaphoreType.DMA,
        pltpu.SemaphoreType.DMA,
    ]


def _mp_sc2(yu0, yu1, rowu, colu, wu, yg0, yg1, rowg, colg, wg):
    half = jax.ShapeDtypeStruct((N, HALF), jnp.float32)
    return pl.kernel(
        _mp_body_2,
        out_type=(half, half, half, half),
        mesh=plsc.VectorSubcoreMesh(**_SC_MESH),
        scratch_types=_mp_scratch(),
    )(yu0, yu1, rowu, colu, wu, yg0, yg1, rowg, colg, wg)


def _mp_sc1(y0, y1, row, col, w):
    half = jax.ShapeDtypeStruct((N, HALF), jnp.float32)
    return pl.kernel(
        _mp_body_1,
        out_type=(half, half),
        mesh=plsc.VectorSubcoreMesh(**_SC_MESH),
        scratch_types=_mp_scratch(),
    )(y0, y1, row, col, w)


def _dis_of(degblk):
    d = degblk + 1.0
    return jnp.where(d > 0, lax.rsqrt(jnp.maximum(d, 1e-12)), 0.0)


def _tc_b_body(deg_ref, embu_ref, wu_ref, embg_ref, wg_ref,
               yu0_ref, yu1_ref, disu_ref, yg0_ref, yg1_ref, disg_ref):
    disu = _dis_of(deg_ref[0][:, 0:1])
    disg = _dis_of(deg_ref[1][:, 0:1])
    yu = disu * jnp.dot(embu_ref[...], wu_ref[...],
                        preferred_element_type=jnp.float32)
    yg = disg * jnp.dot(embg_ref[...], wg_ref[...],
                        preferred_element_type=jnp.float32)
    yu0_ref[...] = yu[:, :HALF]
    yu1_ref[...] = yu[:, HALF:]
    disu_ref[...] = disu
    yg0_ref[...] = yg[:, :HALF]
    yg1_ref[...] = yg[:, HALF:]
    disg_ref[...] = disg


def _tc_b(deg2, emb_u, W_u1, emb_g, W_g1):
    half = jax.ShapeDtypeStruct((N, HALF), jnp.float32)
    dis = jax.ShapeDtypeStruct((N, 1), jnp.float32)
    return pl.pallas_call(
        _tc_b_body,
        grid=(GRID,),
        in_specs=[
            pl.BlockSpec((2, RBLK, HALF), lambda i: (0, i, 0)),
            pl.BlockSpec((RBLK, H), lambda i: (i, 0)),
            pl.BlockSpec((H, H), lambda i: (0, 0)),
            pl.BlockSpec((RBLK, H), lambda i: (i, 0)),
            pl.BlockSpec((H, H), lambda i: (0, 0)),
        ],
        out_specs=[
            pl.BlockSpec((RBLK, HALF), lambda i: (i, 0)),
            pl.BlockSpec((RBLK, HALF), lambda i: (i, 0)),
            pl.BlockSpec((RBLK, 1), lambda i: (i, 0)),
            pl.BlockSpec((RBLK, HALF), lambda i: (i, 0)),
            pl.BlockSpec((RBLK, HALF), lambda i: (i, 0)),
            pl.BlockSpec((RBLK, 1), lambda i: (i, 0)),
        ],
        out_shape=(half, half, dis, half, half, dis),
    )(deg2, emb_u, W_u1, emb_g, W_g1)


def _tc_d_body(au0_ref, au1_ref, yu0_ref, yu1_ref, disu_ref, bu_ref, wu2_ref,
               ag0_ref, ag1_ref, yg0_ref, yg1_ref, disg_ref, bg_ref,
               g_ref, y20_ref, y21_ref):
    disu = disu_ref[...]
    u1 = jnp.concatenate(
        [au0_ref[...] + yu0_ref[...], au1_ref[...] + yu1_ref[...]], axis=1)
    u1 = jax.nn.relu(disu * u1 + bu_ref[...])
    y2 = disu * jnp.dot(u1, wu2_ref[...], preferred_element_type=jnp.float32)
    y20_ref[...] = y2[:, :HALF]
    y21_ref[...] = y2[:, HALF:]
    disg = disg_ref[...]
    g = jnp.concatenate(
        [ag0_ref[...] + yg0_ref[...], ag1_ref[...] + yg1_ref[...]], axis=1)
    g_ref[...] = jax.nn.relu(disg * g + bg_ref[...])


def _tc_d(au0, au1, yu0, yu1, dis_u, b_u1, W_u2, ag0, ag1, yg0, yg1, dis_g,
          b_g1):
    half = jax.ShapeDtypeStruct((N, HALF), jnp.float32)
    full = jax.ShapeDtypeStruct((N, H), jnp.float32)
    rb = lambda i: (i, 0)
    return pl.pallas_call(
        _tc_d_body,
        grid=(GRID,),
        in_specs=[
            pl.BlockSpec((RBLK, HALF), rb), pl.BlockSpec((RBLK, HALF), rb),
            pl.BlockSpec((RBLK, HALF), rb), pl.BlockSpec((RBLK, HALF), rb),
            pl.BlockSpec((RBLK, 1), rb),
            pl.BlockSpec((1, H), lambda i: (0, 0)),
            pl.BlockSpec((H, H), lambda i: (0, 0)),
            pl.BlockSpec((RBLK, HALF), rb), pl.BlockSpec((RBLK, HALF), rb),
            pl.BlockSpec((RBLK, HALF), rb), pl.BlockSpec((RBLK, HALF), rb),
            pl.BlockSpec((RBLK, 1), rb),
            pl.BlockSpec((1, H), lambda i: (0, 0)),
        ],
        out_specs=[
            pl.BlockSpec((RBLK, H), rb),
            pl.BlockSpec((RBLK, HALF), rb), pl.BlockSpec((RBLK, HALF), rb),
        ],
        out_shape=(full, half, half),
    )(au0, au1, yu0, yu1, dis_u, b_u1, W_u2, ag0, ag1, yg0, yg1, dis_g, b_g1)


def _tc_f_body(a0_ref, a1_ref, y0_ref, y1_ref, dis_ref, b_ref, u_ref):
    u = jnp.concatenate(
        [a0_ref[...] + y0_ref[...], a1_ref[...] + y1_ref[...]], axis=1)
    u_ref[...] = dis_ref[...] * u + b_ref[...]


def _tc_f(a0, a1, y0, y1, dis, b):
    rb = lambda i: (i, 0)
    return pl.pallas_call(
        _tc_f_body,
        grid=(GRID,),
        in_specs=[
            pl.BlockSpec((RBLK, HALF), rb), pl.BlockSpec((RBLK, HALF), rb),
            pl.BlockSpec((RBLK, HALF), rb), pl.BlockSpec((RBLK, HALF), rb),
            pl.BlockSpec((RBLK, 1), rb),
            pl.BlockSpec((1, H), lambda i: (0, 0)),
        ],
        out_specs=pl.BlockSpec((RBLK, H), rb),
        out_shape=jax.ShapeDtypeStruct((N, H), jnp.float32),
    )(a0, a1, y0, y1, dis, b)


def kernel(x, user_edge_index, user_edge_weight, group_edge_index,
           group_edge_weight, user_emb, group_emb, W_u1, b_u1, W_u2, b_u2,
           W_g1, b_g1):
    ipad = jnp.zeros((EPAD,), user_edge_index.dtype)
    fpad = jnp.zeros((EPAD,), jnp.float32)
    row_u = jnp.concatenate([user_edge_index[0], ipad]).reshape(EB, BATCH)
    col_u = jnp.concatenate([user_edge_index[1], ipad]).reshape(EB, BATCH)
    row_g = jnp.concatenate([group_edge_index[0], ipad]).reshape(EB, BATCH)
    col_g = jnp.concatenate([group_edge_index[1], ipad]).reshape(EB, BATCH)
    w_u = jnp.concatenate([user_edge_weight, fpad])
    w_g = jnp.concatenate([group_edge_weight, fpad])

    ones_tab = jnp.ones((N, HALF), jnp.float32)
    deg2 = _deg_sc(col_u, w_u, col_g, w_g, ones_tab)
    yu0, yu1, dis_u, yg0, yg1, dis_g = _tc_b(deg2, user_emb, W_u1,
                                             group_emb, W_g1)
    au0, au1, ag0, ag1 = _mp_sc2(yu0, yu1, row_u, col_u, w_u,
                                 yg0, yg1, row_g, col_g, w_g)
    g, y20, y21 = _tc_d(au0, au1, yu0, yu1, dis_u, b_u1.reshape(1, H), W_u2,
                        ag0, ag1, yg0, yg1, dis_g, b_g1.reshape(1, H))
    a20, a21 = _mp_sc1(y20, y21, row_u, col_u, w_u)
    u = _tc_f(a20, a21, y20, y21, dis_u, b_u2.reshape(1, H))
    return (u, g)


# speedup vs baseline: 1.3386x; 1.2818x over previous
import functools

import jax
import jax.numpy as jnp
from jax import lax
from jax.experimental import pallas as pl
from jax.experimental.pallas import tpu as pltpu
from jax.experimental.pallas import tpu_sc as plsc

N = 10000
H = 256
HALF = 128
E = 160000
NS = 16
NC = 2
BATCH = 128
BPT = 80
EB = NS * BPT
EPAD = EB * BATCH - E
CHB = 16
NCH = BPT // CHB
STRIPE = 624
NREM = N - NS * STRIPE
ZROWS = 16
RBLK = 2000
GRID = N // RBLK

_SC_MESH = dict(core_axis_name="c", subcore_axis_name="s")


def _zero_fill(buf, nrows, ncols):
    zeros16 = jnp.zeros((16,), jnp.float32)

    def body(i, _):
        for j in range(ncols // 16):
            buf[i, pl.ds(j * 16, 16)] = zeros16
        return 0

    lax.fori_loop(0, nrows, body, 0)


def _zero_acc(acc, s, zbuf):
    base = s * STRIPE

    def body(k, _):
        pltpu.sync_copy(zbuf, acc.at[pl.ds(base + k * 16, 16)])
        return 0

    lax.fori_loop(0, STRIPE // 16, body, 0)

    @pl.when(s == NS - 1)
    def _():
        pltpu.sync_copy(zbuf, acc.at[pl.ds(NS * STRIPE, NREM)])


def _writeout(acc, out_ref, s):
    base = s * STRIPE
    pltpu.sync_copy(acc.at[pl.ds(base, STRIPE)],
                    out_ref.at[pl.ds(base, STRIPE)])

    @pl.when(s == NS - 1)
    def _():
        pltpu.sync_copy(acc.at[pl.ds(NS * STRIPE, NREM)],
                        out_ref.at[pl.ds(NS * STRIPE, NREM)])


def _graph_pass(gather_tab, row2d, col2d, w1d, out_ref, ncols,
                rowv2, colv2, wv, rbufA, rbufB, zbuf, acc, semA, semB,
                semSA, semSB, s):
    base_b = s * BPT
    plsc.subcore_barrier()

    def scale_rows(rbuf, b):
        boff = b * BATCH

        zeros16 = jnp.zeros((16,), jnp.float32)

        def scale(g, _):
            wvec = wv[pl.ds(boff + g * 16, 16)]
            for t in range(16):
                e = g * 16 + t
                wrow = zeros16 + wvec[t]
                for j in range(ncols // 16):
                    sl = pl.ds(j * 16, 16)
                    rbuf[e, sl] = rbuf[e, sl] * wrow
            return 0

        lax.fori_loop(0, BATCH // 16, scale, 0)

    def chunk(ch, _):
        cb = base_b + ch * CHB
        pltpu.sync_copy(row2d.at[pl.ds(cb, CHB)], rowv2)
        pltpu.sync_copy(col2d.at[pl.ds(cb, CHB)], colv2)
        pltpu.sync_copy(w1d.at[pl.ds(cb * BATCH, CHB * BATCH)], wv)

        pltpu.async_copy(gather_tab.at[rowv2.at[0]], rbufA, semA)

        def pair(i, _):
            b0 = 2 * i
            pltpu.async_copy(gather_tab.at[rowv2.at[b0 + 1]], rbufB, semB)
            pltpu.make_async_copy(gather_tab.at[rowv2.at[b0]],
                                  rbufA, semA).wait()
            scale_rows(rbufA, b0)
            pltpu.sync_copy(rbufA, acc.at[colv2.at[b0]], add=True)
            nxt = jnp.minimum(b0 + 2, CHB - 1)
            pltpu.async_copy(gather_tab.at[rowv2.at[nxt]], rbufA, semA)
            pltpu.make_async_copy(gather_tab.at[rowv2.at[b0 + 1]],
                                  rbufB, semB).wait()
            scale_rows(rbufB, b0 + 1)
            pltpu.sync_copy(rbufB, acc.at[colv2.at[b0 + 1]], add=True)
            return 0

        lax.fori_loop(0, CHB // 2, pair, 0)
        pltpu.make_async_copy(gather_tab.at[rowv2.at[0]], rbufA, semA).wait()
        return 0

    lax.fori_loop(0, NCH, chunk, 0)

    plsc.subcore_barrier()
    _writeout(acc, out_ref, s)
    plsc.subcore_barrier()


def _deg_fill_pass(col2d, w1d, out_ref, colv2, wv, rbuf, acc, s):
    base_b = s * BPT

    def chunk(ch, _):
        cb = base_b + ch * CHB
        pltpu.sync_copy(col2d.at[pl.ds(cb, CHB)], colv2)
        pltpu.sync_copy(w1d.at[pl.ds(cb * BATCH, CHB * BATCH)], wv)

        def bstage(b, _):
            boff = b * BATCH
            zeros16 = jnp.zeros((16,), jnp.float32)

            def fill(g, _):
                wvec = wv[pl.ds(boff + g * 16, 16)]
                for t in range(16):
                    e = g * 16 + t
                    wrow = zeros16 + wvec[t]
                    for j in range(HALF // 16):
                        sl = pl.ds(j * 16, 16)
                        rbuf[e, sl] = rbuf[e, sl] * 0.0 + wrow
                return 0

            lax.fori_loop(0, BATCH // 16, fill, 0)
            pltpu.sync_copy(rbuf, acc.at[colv2.at[b]], add=True)
            return 0

        lax.fori_loop(0, CHB, bstage, 0)
        return 0

    lax.fori_loop(0, NCH, chunk, 0)

    plsc.subcore_barrier()
    _writeout(acc, out_ref, s)
    plsc.subcore_barrier()


def _deg_body(colu2d, wu1d, colg2d, wg1d, out,
              rowv2, colv2, wv, rbufA, rbufB, zbuf, acc, semA, semB,
              semSA, semSB):
    c = lax.axis_index("c")
    s = lax.axis_index("s")
    _zero_fill(zbuf, ZROWS, HALF)
    _zero_fill(rbufA, BATCH, HALF)
    _zero_acc(acc, s, zbuf)

    @pl.when(c == 0)
    def _():
        _deg_fill_pass(colu2d, wu1d, out.at[0], colv2, wv, rbufA, acc, s)

    @pl.when(c == 1)
    def _():
        _deg_fill_pass(colg2d, wg1d, out.at[1], colv2, wv, rbufA, acc, s)


def _deg_sc(col_u2d, w_u1d, col_g2d, w_g1d):
    return pl.kernel(
        _deg_body,
        out_type=jax.ShapeDtypeStruct((2, N, HALF), jnp.float32),
        mesh=plsc.VectorSubcoreMesh(**_SC_MESH),
        scratch_types=_mp_scratch(),
    )(col_u2d, w_u1d, col_g2d, w_g1d)


def _mp_core(y0, y1, row2d, col2d, w1d, out0, out1,
             rowv2, colv2, wv, rbufA, rbufB, zbuf, acc, semA, semB,
             semSA, semSB, c, s):
    _zero_acc(acc, s, zbuf)

    @pl.when(c == 0)
    def _():
        _graph_pass(y0, row2d, col2d, w1d, out0, HALF,
                    rowv2, colv2, wv, rbufA, rbufB, zbuf, acc, semA, semB,
                    semSA, semSB, s)

    @pl.when(c == 1)
    def _():
        _graph_pass(y1, row2d, col2d, w1d, out1, HALF,
                    rowv2, colv2, wv, rbufA, rbufB, zbuf, acc, semA, semB,
                    semSA, semSB, s)


def _mp_body_2(yu0, yu1, rowu, colu, wu, yg0, yg1, rowg, colg, wg,
               ou0, ou1, og0, og1, rowv2, colv2, wv, rbufA, rbufB, zbuf,
               acc, semA, semB, semSA, semSB):
    c = lax.axis_index("c")
    s = lax.axis_index("s")
    _zero_fill(zbuf, ZROWS, HALF)
    _mp_core(yu0, yu1, rowu, colu, wu, ou0, ou1,
             rowv2, colv2, wv, rbufA, rbufB, zbuf, acc, semA, semB,
             semSA, semSB, c, s)
    _mp_core(yg0, yg1, rowg, colg, wg, og0, og1,
             rowv2, colv2, wv, rbufA, rbufB, zbuf, acc, semA, semB,
             semSA, semSB, c, s)


def _mp_body_1(yu0, yu1, rowu, colu, wu, ou0, ou1,
               rowv2, colv2, wv, rbufA, rbufB, zbuf, acc, semA, semB,
               semSA, semSB):
    c = lax.axis_index("c")
    s = lax.axis_index("s")
    _zero_fill(zbuf, ZROWS, HALF)
    _mp_core(yu0, yu1, rowu, colu, wu, ou0, ou1,
             rowv2, colv2, wv, rbufA, rbufB, zbuf, acc, semA, semB,
             semSA, semSB, c, s)


def _mp_scratch():
    return [
        pltpu.VMEM((CHB, BATCH), jnp.int32),
        pltpu.VMEM((CHB, BATCH), jnp.int32),
        pltpu.VMEM((CHB * BATCH,), jnp.float32),
        pltpu.VMEM((BATCH, HALF), jnp.float32),
        pltpu.VMEM((BATCH, HALF), jnp.float32),
        pltpu.VMEM((ZROWS, HALF), jnp.float32),
        pltpu.VMEM_SHARED((N, HALF), jnp.float32),
        pltpu.SemaphoreType.DMA,
        pltpu.SemaphoreType.DMA,
        pltpu.SemaphoreType.DMA,
        pltpu.SemaphoreType.DMA,
    ]


def _mp_sc2(yu0, yu1, rowu, colu, wu, yg0, yg1, rowg, colg, wg):
    half = jax.ShapeDtypeStruct((N, HALF), jnp.float32)
    return pl.kernel(
        _mp_body_2,
        out_type=(half, half, half, half),
        mesh=plsc.VectorSubcoreMesh(**_SC_MESH),
        scratch_types=_mp_scratch(),
    )(yu0, yu1, rowu, colu, wu, yg0, yg1, rowg, colg, wg)


def _mp_sc1(y0, y1, row, col, w):
    half = jax.ShapeDtypeStruct((N, HALF), jnp.float32)
    return pl.kernel(
        _mp_body_1,
        out_type=(half, half),
        mesh=plsc.VectorSubcoreMesh(**_SC_MESH),
        scratch_types=_mp_scratch(),
    )(y0, y1, row, col, w)


def _dis_of(degblk):
    d = degblk + 1.0
    return jnp.where(d > 0, lax.rsqrt(jnp.maximum(d, 1e-12)), 0.0)


def _tc_b_body(deg_ref, embu_ref, wu_ref, embg_ref, wg_ref,
               yu0_ref, yu1_ref, disu_ref, yg0_ref, yg1_ref, disg_ref):
    disu = _dis_of(deg_ref[0][:, 0:1])
    disg = _dis_of(deg_ref[1][:, 0:1])
    yu = disu * jnp.dot(embu_ref[...], wu_ref[...],
                        preferred_element_type=jnp.float32)
    yg = disg * jnp.dot(embg_ref[...], wg_ref[...],
                        preferred_element_type=jnp.float32)
    yu0_ref[...] = yu[:, :HALF]
    yu1_ref[...] = yu[:, HALF:]
    disu_ref[...] = disu
    yg0_ref[...] = yg[:, :HALF]
    yg1_ref[...] = yg[:, HALF:]
    disg_ref[...] = disg


def _tc_b(deg2, emb_u, W_u1, emb_g, W_g1):
    half = jax.ShapeDtypeStruct((N, HALF), jnp.float32)
    dis = jax.ShapeDtypeStruct((N, 1), jnp.float32)
    return pl.pallas_call(
        _tc_b_body,
        grid=(GRID,),
        in_specs=[
            pl.BlockSpec((2, RBLK, HALF), lambda i: (0, i, 0)),
            pl.BlockSpec((RBLK, H), lambda i: (i, 0)),
            pl.BlockSpec((H, H), lambda i: (0, 0)),
            pl.BlockSpec((RBLK, H), lambda i: (i, 0)),
            pl.BlockSpec((H, H), lambda i: (0, 0)),
        ],
        out_specs=[
            pl.BlockSpec((RBLK, HALF), lambda i: (i, 0)),
            pl.BlockSpec((RBLK, HALF), lambda i: (i, 0)),
            pl.BlockSpec((RBLK, 1), lambda i: (i, 0)),
            pl.BlockSpec((RBLK, HALF), lambda i: (i, 0)),
            pl.BlockSpec((RBLK, HALF), lambda i: (i, 0)),
            pl.BlockSpec((RBLK, 1), lambda i: (i, 0)),
        ],
        out_shape=(half, half, dis, half, half, dis),
    )(deg2, emb_u, W_u1, emb_g, W_g1)


def _tc_d_body(au0_ref, au1_ref, yu0_ref, yu1_ref, disu_ref, bu_ref, wu2_ref,
               ag0_ref, ag1_ref, yg0_ref, yg1_ref, disg_ref, bg_ref,
               g_ref, y20_ref, y21_ref):
    disu = disu_ref[...]
    u1 = jnp.concatenate(
        [au0_ref[...] + yu0_ref[...], au1_ref[...] + yu1_ref[...]], axis=1)
    u1 = jax.nn.relu(disu * u1 + bu_ref[...])
    y2 = disu * jnp.dot(u1, wu2_ref[...], preferred_element_type=jnp.float32)
    y20_ref[...] = y2[:, :HALF]
    y21_ref[...] = y2[:, HALF:]
    disg = disg_ref[...]
    g = jnp.concatenate(
        [ag0_ref[...] + yg0_ref[...], ag1_ref[...] + yg1_ref[...]], axis=1)
    g_ref[...] = jax.nn.relu(disg * g + bg_ref[...])


def _tc_d(au0, au1, yu0, yu1, dis_u, b_u1, W_u2, ag0, ag1, yg0, yg1, dis_g,
          b_g1):
    half = jax.ShapeDtypeStruct((N, HALF), jnp.float32)
    full = jax.ShapeDtypeStruct((N, H), jnp.float32)
    rb = lambda i: (i, 0)
    return pl.pallas_call(
        _tc_d_body,
        grid=(GRID,),
        in_specs=[
            pl.BlockSpec((RBLK, HALF), rb), pl.BlockSpec((RBLK, HALF), rb),
            pl.BlockSpec((RBLK, HALF), rb), pl.BlockSpec((RBLK, HALF), rb),
            pl.BlockSpec((RBLK, 1), rb),
            pl.BlockSpec((1, H), lambda i: (0, 0)),
            pl.BlockSpec((H, H), lambda i: (0, 0)),
            pl.BlockSpec((RBLK, HALF), rb), pl.BlockSpec((RBLK, HALF), rb),
            pl.BlockSpec((RBLK, HALF), rb), pl.BlockSpec((RBLK, HALF), rb),
            pl.BlockSpec((RBLK, 1), rb),
            pl.BlockSpec((1, H), lambda i: (0, 0)),
        ],
        out_specs=[
            pl.BlockSpec((RBLK, H), rb),
            pl.BlockSpec((RBLK, HALF), rb), pl.BlockSpec((RBLK, HALF), rb),
        ],
        out_shape=(full, half, half),
    )(au0, au1, yu0, yu1, dis_u, b_u1, W_u2, ag0, ag1, yg0, yg1, dis_g, b_g1)


def _tc_f_body(a0_ref, a1_ref, y0_ref, y1_ref, dis_ref, b_ref, u_ref):
    u = jnp.concatenate(
        [a0_ref[...] + y0_ref[...], a1_ref[...] + y1_ref[...]], axis=1)
    u_ref[...] = dis_ref[...] * u + b_ref[...]


def _tc_f(a0, a1, y0, y1, dis, b):
    rb = lambda i: (i, 0)
    return pl.pallas_call(
        _tc_f_body,
        grid=(GRID,),
        in_specs=[
            pl.BlockSpec((RBLK, HALF), rb), pl.BlockSpec((RBLK, HALF), rb),
            pl.BlockSpec((RBLK, HALF), rb), pl.BlockSpec((RBLK, HALF), rb),
            pl.BlockSpec((RBLK, 1), rb),
            pl.BlockSpec((1, H), lambda i: (0, 0)),
        ],
        out_specs=pl.BlockSpec((RBLK, H), rb),
        out_shape=jax.ShapeDtypeStruct((N, H), jnp.float32),
    )(a0, a1, y0, y1, dis, b)


def kernel(x, user_edge_index, user_edge_weight, group_edge_index,
           group_edge_weight, user_emb, group_emb, W_u1, b_u1, W_u2, b_u2,
           W_g1, b_g1):
    ipad = jnp.zeros((EPAD,), user_edge_index.dtype)
    fpad = jnp.zeros((EPAD,), jnp.float32)
    row_u = jnp.concatenate([user_edge_index[0], ipad]).reshape(EB, BATCH)
    col_u = jnp.concatenate([user_edge_index[1], ipad]).reshape(EB, BATCH)
    row_g = jnp.concatenate([group_edge_index[0], ipad]).reshape(EB, BATCH)
    col_g = jnp.concatenate([group_edge_index[1], ipad]).reshape(EB, BATCH)
    w_u = jnp.concatenate([user_edge_weight, fpad])
    w_g = jnp.concatenate([group_edge_weight, fpad])

    deg2 = _deg_sc(col_u, w_u, col_g, w_g)
    yu0, yu1, dis_u, yg0, yg1, dis_g = _tc_b(deg2, user_emb, W_u1,
                                             group_emb, W_g1)
    au0, au1, ag0, ag1 = _mp_sc2(yu0, yu1, row_u, col_u, w_u,
                                 yg0, yg1, row_g, col_g, w_g)
    g, y20, y21 = _tc_d(au0, au1, yu0, yu1, dis_u, b_u1.reshape(1, H), W_u2,
                        ag0, ag1, yg0, yg1, dis_g, b_g1.reshape(1, H))
    a20, a21 = _mp_sc1(y20, y21, row_u, col_u, w_u)
    u = _tc_f(a20, a21, y20, y21, dis_u, b_u2.reshape(1, H))
    return (u, g)
